# Initial kernel scaffold; baseline (speedup 1.0000x reference)
import jax
import jax.numpy as jnp
from jax.experimental import pallas as pl


def kernel(z, pos, batch, edge_index, emb, mlp_W1, mlp_b1, mlp_W2, mlp_b2, lin1_W, lin2_W, lin2_b, lin_W, lin_b, pool_W, pool_b):
    raise NotImplementedError("write your pallas kernel here")


# speedup vs baseline: 2.2694x; 2.2694x over previous
import functools
import math

import jax
import jax.numpy as jnp
from jax import lax
from jax.experimental import pallas as pl
from jax.experimental.pallas import tpu as pltpu
from jax.experimental.pallas import tpu_sc as plsc

N = 10000
E = 320000
H = 128
NG = 50
NGP = 64
L = 6
CUTOFF = 10.0
NGRAPH = 16
VOCABP = 128
NP_ = 10240

NC = 2
NS = 16
NW = NC * NS
EW = E // NW
CH = 80
NCH = EW // CH
RPT = NP_ // NS

_DELTA = CUTOFF / (NG - 1)
_COEFF = -0.5 / (_DELTA * _DELTA)
_LOG2 = math.log(2.0)


def _ssp(x):
    return jnp.maximum(x, 0.0) + jnp.log(1.0 + jnp.exp(-jnp.abs(x))) - _LOG2


def _k1_body(px_h, py_h, pz_h, row_h, col_h, d2_h,
             px_v, py_v, pz_v, row_v, col_v, d2_v):
    c = lax.axis_index("c")
    s = lax.axis_index("s")
    wid = s * NC + c
    base = wid * EW
    pltpu.sync_copy(px_h, px_v)
    pltpu.sync_copy(py_h, py_v)
    pltpu.sync_copy(pz_h, pz_v)
    pltpu.sync_copy(row_h.at[pl.ds(base, EW)], row_v)
    pltpu.sync_copy(col_h.at[pl.ds(base, EW)], col_v)

    def body(i, carry):
        sl = pl.ds(i * 16, 16)
        r = row_v[sl]
        cc = col_v[sl]
        dx = plsc.load_gather(px_v, [r]) - plsc.load_gather(px_v, [cc])
        dy = plsc.load_gather(py_v, [r]) - plsc.load_gather(py_v, [cc])
        dz = plsc.load_gather(pz_v, [r]) - plsc.load_gather(pz_v, [cc])
        d2_v[sl] = dx * dx + dy * dy + dz * dz
        return carry

    lax.fori_loop(0, EW // 16, body, 0)
    pltpu.sync_copy(d2_v, d2_h.at[pl.ds(base, EW)])


_k1 = functools.partial(
    pl.kernel,
    mesh=plsc.VectorSubcoreMesh(core_axis_name="c", subcore_axis_name="s"),
    compiler_params=pltpu.CompilerParams(needs_layout_passes=False),
    out_type=jax.ShapeDtypeStruct((E,), jnp.float32),
    scratch_types=[
        pltpu.VMEM((N,), jnp.float32),
        pltpu.VMEM((N,), jnp.float32),
        pltpu.VMEM((N,), jnp.float32),
        pltpu.VMEM((EW,), jnp.int32),
        pltpu.VMEM((EW,), jnp.int32),
        pltpu.VMEM((EW,), jnp.float32),
    ],
)(_k1_body)


def _k3_body(xl_h, wf_h, row_h, col_h, zeros_h, out_h,
             agg_sp, row_v, col_v, rows_v, wf_v, sem):
    c = lax.axis_index("c")
    s = lax.axis_index("s")
    wid = s * NC + c
    base = wid * EW
    pltpu.sync_copy(zeros_h.at[pl.ds(s * RPT, RPT)],
                    agg_sp.at[pl.ds(s * RPT, RPT)])
    plsc.subcore_barrier()

    def chunk(i, carry):
        off = base + i * CH
        pltpu.sync_copy(row_h.at[pl.ds(off, CH)], row_v)
        pltpu.sync_copy(col_h.at[pl.ds(off, CH)], col_v)
        pltpu.async_copy(xl_h.at[row_v], rows_v, sem).wait()
        pltpu.sync_copy(wf_h.at[pl.ds(off, CH)], wf_v)

        def mrow(r, carry2):
            for j in range(8):
                sl = pl.ds(j * 16, 16)
                rows_v[r, sl] = rows_v[r, sl] * wf_v[r, sl]
            return carry2

        lax.fori_loop(0, CH, mrow, 0)
        pltpu.sync_copy(rows_v, agg_sp.at[col_v], add=True)
        return carry

    lax.fori_loop(0, NCH, chunk, 0)
    plsc.subcore_barrier()
    pltpu.sync_copy(agg_sp.at[pl.ds(s * RPT, RPT)],
                    out_h.at[c, pl.ds(s * RPT, RPT)])


_k3 = functools.partial(
    pl.kernel,
    mesh=plsc.VectorSubcoreMesh(core_axis_name="c", subcore_axis_name="s"),
    out_type=jax.ShapeDtypeStruct((NC, NP_, H), jnp.float32),
    scratch_types=[
        pltpu.VMEM_SHARED((NP_, H), jnp.float32),
        pltpu.VMEM((CH,), jnp.int32),
        pltpu.VMEM((CH,), jnp.int32),
        pltpu.VMEM((CH, H), jnp.float32),
        pltpu.VMEM((CH, H), jnp.float32),
        pltpu.SemaphoreType.DMA,
    ],
)(_k3_body)


BE = 2560


def _k2_body(d2_ref, w1_ref, b1_ref, w2_ref, b2_ref, wf_ref):
    d2 = d2_ref[...]
    d = jnp.sqrt(d2 + 1e-12)
    offs = _DELTA * lax.broadcasted_iota(jnp.int32, (1, NGP), 1).astype(jnp.float32)
    g = jnp.exp(_COEFF * (d - offs) ** 2)
    t = jnp.dot(g, w1_ref[...], preferred_element_type=jnp.float32) + b1_ref[...]
    t = _ssp(t)
    t = jnp.dot(t, w2_ref[...], preferred_element_type=jnp.float32) + b2_ref[...]
    cut = 0.5 * (jnp.cos(d * (math.pi / CUTOFF)) + 1.0)
    wf_ref[...] = t * cut


def _k2(d2c, w1p, b1, w2, b2):
    return pl.pallas_call(
        _k2_body,
        grid=(E // BE,),
        in_specs=[
            pl.BlockSpec((BE, 1), lambda i: (i, 0)),
            pl.BlockSpec((NGP, H), lambda i: (0, 0)),
            pl.BlockSpec((1, H), lambda i: (0, 0)),
            pl.BlockSpec((H, H), lambda i: (0, 0)),
            pl.BlockSpec((1, H), lambda i: (0, 0)),
        ],
        out_specs=pl.BlockSpec((BE, H), lambda i: (i, 0)),
        out_shape=jax.ShapeDtypeStruct((E, H), jnp.float32),
    )(d2c, w1p, b1, w2, b2)


BN = 2560


def _k0_body(z_ref, emb_ref, lin1_ref, h_ref, xl_ref):
    zb = z_ref[...]
    oh = (zb == lax.broadcasted_iota(jnp.int32, (1, VOCABP), 1)).astype(jnp.float32)
    h = jnp.dot(oh, emb_ref[...], preferred_element_type=jnp.float32)
    h_ref[...] = h
    xl_ref[...] = jnp.dot(h, lin1_ref[...], preferred_element_type=jnp.float32)


def _k0(z_p, embp, lin1_0):
    return pl.pallas_call(
        _k0_body,
        grid=(NP_ // BN,),
        in_specs=[
            pl.BlockSpec((BN, 1), lambda i: (i, 0)),
            pl.BlockSpec((VOCABP, H), lambda i: (0, 0)),
            pl.BlockSpec((H, H), lambda i: (0, 0)),
        ],
        out_specs=[
            pl.BlockSpec((BN, H), lambda i: (i, 0)),
            pl.BlockSpec((BN, H), lambda i: (i, 0)),
        ],
        out_shape=[
            jax.ShapeDtypeStruct((NP_, H), jnp.float32),
            jax.ShapeDtypeStruct((NP_, H), jnp.float32),
        ],
    )(z_p, embp, lin1_0)


def _k4_body(a0_ref, a1_ref, h_ref, l2w_ref, l2b_ref, lw_ref, lb_ref,
             l1n_ref, hout_ref, xlout_ref):
    agg = a0_ref[0] + a1_ref[0]
    t = _ssp(jnp.dot(agg, l2w_ref[...], preferred_element_type=jnp.float32)
             + l2b_ref[...])
    x2 = jnp.dot(t, lw_ref[...], preferred_element_type=jnp.float32) + lb_ref[...]
    hn = h_ref[...] + x2
    hout_ref[...] = hn
    xlout_ref[...] = jnp.dot(hn, l1n_ref[...], preferred_element_type=jnp.float32)


def _k4(aggp, h, l2w, l2b, lw, lb, l1n):
    return pl.pallas_call(
        _k4_body,
        grid=(NP_ // BN,),
        in_specs=[
            pl.BlockSpec((1, BN, H), lambda i: (0, i, 0)),
            pl.BlockSpec((1, BN, H), lambda i: (1, i, 0)),
            pl.BlockSpec((BN, H), lambda i: (i, 0)),
            pl.BlockSpec((H, H), lambda i: (0, 0)),
            pl.BlockSpec((1, H), lambda i: (0, 0)),
            pl.BlockSpec((H, H), lambda i: (0, 0)),
            pl.BlockSpec((1, H), lambda i: (0, 0)),
            pl.BlockSpec((H, H), lambda i: (0, 0)),
        ],
        out_specs=[
            pl.BlockSpec((BN, H), lambda i: (i, 0)),
            pl.BlockSpec((BN, H), lambda i: (i, 0)),
        ],
        out_shape=[
            jax.ShapeDtypeStruct((NP_, H), jnp.float32),
            jax.ShapeDtypeStruct((NP_, H), jnp.float32),
        ],
    )(aggp, aggp, h, l2w, l2b, lw, lb, l1n)


def _k5_body(b_ref, h_ref, pw_ref, pb_ref, out_ref, sums, counts):
    i = pl.program_id(0)

    @pl.when(i == 0)
    def _init():
        sums[...] = jnp.zeros_like(sums)
        counts[...] = jnp.zeros_like(counts)

    bb = b_ref[...]
    oh = (bb == lax.broadcasted_iota(jnp.int32, (1, NGRAPH), 1)).astype(jnp.float32)
    hb = h_ref[...]
    dn = (((0,), (0,)), ((), ()))
    sums[...] += lax.dot_general(oh, hb, dn, preferred_element_type=jnp.float32)
    counts[...] += lax.dot_general(oh, jnp.ones_like(hb), dn,
                                   preferred_element_type=jnp.float32)

    @pl.when(i == NP_ // BN - 1)
    def _fin():
        pooled = sums[...] / jnp.maximum(counts[...], 1.0)
        out_ref[...] = (jnp.dot(pooled, pw_ref[...],
                                preferred_element_type=jnp.float32) + pb_ref[...])


def _k5(batch_p, h, pool_W, pool_b):
    return pl.pallas_call(
        _k5_body,
        grid=(NP_ // BN,),
        in_specs=[
            pl.BlockSpec((BN, 1), lambda i: (i, 0)),
            pl.BlockSpec((BN, H), lambda i: (i, 0)),
            pl.BlockSpec((H, H), lambda i: (0, 0)),
            pl.BlockSpec((1, H), lambda i: (0, 0)),
        ],
        out_specs=pl.BlockSpec((NGRAPH, H), lambda i: (0, 0)),
        out_shape=jax.ShapeDtypeStruct((NGRAPH, H), jnp.float32),
        scratch_shapes=[
            pltpu.VMEM((NGRAPH, H), jnp.float32),
            pltpu.VMEM((NGRAPH, H), jnp.float32),
        ],
    )(batch_p, h, pool_W, pool_b)


def kernel(z, pos, batch, edge_index, emb, mlp_W1, mlp_b1, mlp_W2, mlp_b2,
           lin1_W, lin2_W, lin2_b, lin_W, lin_b, pool_W, pool_b):
    row = edge_index[0].astype(jnp.int32)
    col = edge_index[1].astype(jnp.int32)
    posf = pos.astype(jnp.float32)
    px = posf[:, 0] + 0.0
    py = posf[:, 1] + 0.0
    pz = posf[:, 2] + 0.0

    z_p = jnp.pad(z.astype(jnp.int32), (0, NP_ - N)).reshape(NP_, 1)
    batch_p = jnp.pad(batch.astype(jnp.int32), (0, NP_ - N),
                      constant_values=NGRAPH).reshape(NP_, 1)
    embp = jnp.pad(emb, ((0, VOCABP - emb.shape[0]), (0, 0)))
    w1p = jnp.pad(mlp_W1, ((0, 0), (0, NGP - NG), (0, 0)))
    zeros = jnp.zeros((NP_, H), jnp.float32)

    d2 = _k1(px, py, pz, row, col)
    d2c = d2.reshape(E, 1)

    h, xl = _k0(z_p, embp, lin1_W[0])
    for i in range(L):
        wf = _k2(d2c, w1p[i], mlp_b1[i].reshape(1, H),
                 mlp_W2[i], mlp_b2[i].reshape(1, H))
        aggp = _k3(xl, wf, row, col, zeros)
        h, xl = _k4(aggp, h, lin2_W[i], lin2_b[i].reshape(1, H),
                    lin_W[i], lin_b[i].reshape(1, H), lin1_W[(i + 1) % L])

    return _k5(batch_p, h, pool_W, pool_b.reshape(1, H))


# speedup vs baseline: 2.9072x; 1.2810x over previous
import functools
import math

import jax
import jax.numpy as jnp
from jax import lax
from jax.experimental import pallas as pl
from jax.experimental.pallas import tpu as pltpu
from jax.experimental.pallas import tpu_sc as plsc

N = 10000
E = 320000
H = 128
NG = 50
NGP = 64
L = 6
CUTOFF = 10.0
NGRAPH = 16
VOCABP = 128
NP_ = 10240

NC = 2
NS = 16
NW = NC * NS
EW = E // NW
CH = 80
NCH = EW // CH
RPT = NP_ // NS

_DELTA = CUTOFF / (NG - 1)
_COEFF = -0.5 / (_DELTA * _DELTA)
_LOG2 = math.log(2.0)


def _ssp(x):
    return jnp.maximum(x, 0.0) + jnp.log(1.0 + jnp.exp(-jnp.abs(x))) - _LOG2


def _k1_body(px_h, py_h, pz_h, row_h, col_h, d2_h,
             px_v, py_v, pz_v, row_v, col_v, d2_v):
    c = lax.axis_index("c")
    s = lax.axis_index("s")
    wid = s * NC + c
    base = wid * EW
    pltpu.sync_copy(px_h, px_v)
    pltpu.sync_copy(py_h, py_v)
    pltpu.sync_copy(pz_h, pz_v)
    pltpu.sync_copy(row_h.at[pl.ds(base, EW)], row_v)
    pltpu.sync_copy(col_h.at[pl.ds(base, EW)], col_v)

    def body(i, carry):
        sl = pl.ds(i * 16, 16)
        r = row_v[sl]
        cc = col_v[sl]
        dx = plsc.load_gather(px_v, [r]) - plsc.load_gather(px_v, [cc])
        dy = plsc.load_gather(py_v, [r]) - plsc.load_gather(py_v, [cc])
        dz = plsc.load_gather(pz_v, [r]) - plsc.load_gather(pz_v, [cc])
        d2_v[sl] = dx * dx + dy * dy + dz * dz
        return carry

    lax.fori_loop(0, EW // 16, body, 0)
    pltpu.sync_copy(d2_v, d2_h.at[pl.ds(base, EW)])


_k1 = functools.partial(
    pl.kernel,
    mesh=plsc.VectorSubcoreMesh(core_axis_name="c", subcore_axis_name="s"),
    compiler_params=pltpu.CompilerParams(needs_layout_passes=False),
    out_type=jax.ShapeDtypeStruct((E,), jnp.float32),
    scratch_types=[
        pltpu.VMEM((N,), jnp.float32),
        pltpu.VMEM((N,), jnp.float32),
        pltpu.VMEM((N,), jnp.float32),
        pltpu.VMEM((EW,), jnp.int32),
        pltpu.VMEM((EW,), jnp.int32),
        pltpu.VMEM((EW,), jnp.float32),
    ],
)(_k1_body)


def _k3_body(xl_h, wf_h, row_h, col_h, zeros_h, out_h,
             agg_sp, row_v, col_v, rows_v, wf_v, sem):
    c = lax.axis_index("c")
    s = lax.axis_index("s")
    wid = s * NC + c
    base = wid * EW
    pltpu.sync_copy(zeros_h.at[pl.ds(s * RPT, RPT)],
                    agg_sp.at[pl.ds(s * RPT, RPT)])
    plsc.subcore_barrier()

    def chunk(i, carry):
        off = base + i * CH
        pltpu.sync_copy(row_h.at[pl.ds(off, CH)], row_v)
        pltpu.sync_copy(col_h.at[pl.ds(off, CH)], col_v)
        pltpu.async_copy(xl_h.at[row_v], rows_v, sem).wait()
        pltpu.sync_copy(wf_h.at[pl.ds(off, CH)], wf_v)

        def mrow(r, carry2):
            for j in range(8):
                sl = pl.ds(j * 16, 16)
                rows_v[r, sl] = rows_v[r, sl] * wf_v[r, sl]
            return carry2

        lax.fori_loop(0, CH, mrow, 0)
        pltpu.sync_copy(rows_v, agg_sp.at[col_v], add=True)
        return carry

    lax.fori_loop(0, NCH, chunk, 0)
    plsc.subcore_barrier()
    pltpu.sync_copy(agg_sp.at[pl.ds(s * RPT, RPT)],
                    out_h.at[c, pl.ds(s * RPT, RPT)])


_k3 = functools.partial(
    pl.kernel,
    mesh=plsc.VectorSubcoreMesh(core_axis_name="c", subcore_axis_name="s"),
    out_type=jax.ShapeDtypeStruct((NC, NP_, H), jnp.float32),
    scratch_types=[
        pltpu.VMEM_SHARED((NP_, H), jnp.float32),
        pltpu.VMEM((CH,), jnp.int32),
        pltpu.VMEM((CH,), jnp.int32),
        pltpu.VMEM((CH, H), jnp.float32),
        pltpu.VMEM((CH, H), jnp.float32),
        pltpu.SemaphoreType.DMA,
    ],
)(_k3_body)


ER = E // H
BR = 2500


def _k1b_body(d2_ref, d_ref, cut_ref):
    d2 = d2_ref[...]
    d = jnp.sqrt(d2 + 1e-12)
    d_ref[...] = d
    cut_ref[...] = 0.5 * (jnp.cos(d * (math.pi / CUTOFF)) + 1.0)


def _k1b(d2m):
    return pl.pallas_call(
        _k1b_body,
        grid=(ER // BR,),
        in_specs=[pl.BlockSpec((BR, H), lambda i: (i, 0))],
        out_specs=[
            pl.BlockSpec((BR, H), lambda i: (i, 0)),
            pl.BlockSpec((BR, H), lambda i: (i, 0)),
        ],
        out_shape=[
            jax.ShapeDtypeStruct((ER, H), jnp.float32),
            jax.ShapeDtypeStruct((ER, H), jnp.float32),
        ],
    )(d2m)


BE = 2560


def _k2_body(d_ref, cut_ref, w1_ref, b1_ref, w2_ref, b2_ref, wf_ref):
    d = d_ref[...]
    offs = _DELTA * lax.broadcasted_iota(jnp.int32, (1, NGP), 1).astype(jnp.float32)
    g = jnp.exp(_COEFF * (d - offs) ** 2)
    t = jnp.dot(g, w1_ref[...], preferred_element_type=jnp.float32) + b1_ref[...]
    t = _ssp(t)
    t = jnp.dot(t, w2_ref[...], preferred_element_type=jnp.float32) + b2_ref[...]
    wf_ref[...] = t * cut_ref[...]


def _k2(dc, cutc, w1p, b1, w2, b2):
    return pl.pallas_call(
        _k2_body,
        grid=(E // BE,),
        in_specs=[
            pl.BlockSpec((BE, 1), lambda i: (i, 0)),
            pl.BlockSpec((BE, 1), lambda i: (i, 0)),
            pl.BlockSpec((NGP, H), lambda i: (0, 0)),
            pl.BlockSpec((1, H), lambda i: (0, 0)),
            pl.BlockSpec((H, H), lambda i: (0, 0)),
            pl.BlockSpec((1, H), lambda i: (0, 0)),
        ],
        out_specs=pl.BlockSpec((BE, H), lambda i: (i, 0)),
        out_shape=jax.ShapeDtypeStruct((E, H), jnp.float32),
    )(dc, cutc, w1p, b1, w2, b2)


BN = 2560


def _k0_body(z_ref, emb_ref, lin1_ref, h_ref, xl_ref):
    zb = z_ref[...]
    oh = (zb == lax.broadcasted_iota(jnp.int32, (1, VOCABP), 1)).astype(jnp.float32)
    h = jnp.dot(oh, emb_ref[...], preferred_element_type=jnp.float32)
    h_ref[...] = h
    xl_ref[...] = jnp.dot(h, lin1_ref[...], preferred_element_type=jnp.float32)


def _k0(z_p, embp, lin1_0):
    return pl.pallas_call(
        _k0_body,
        grid=(NP_ // BN,),
        in_specs=[
            pl.BlockSpec((BN, 1), lambda i: (i, 0)),
            pl.BlockSpec((VOCABP, H), lambda i: (0, 0)),
            pl.BlockSpec((H, H), lambda i: (0, 0)),
        ],
        out_specs=[
            pl.BlockSpec((BN, H), lambda i: (i, 0)),
            pl.BlockSpec((BN, H), lambda i: (i, 0)),
        ],
        out_shape=[
            jax.ShapeDtypeStruct((NP_, H), jnp.float32),
            jax.ShapeDtypeStruct((NP_, H), jnp.float32),
        ],
    )(z_p, embp, lin1_0)


def _k4_body(a0_ref, a1_ref, h_ref, l2w_ref, l2b_ref, lw_ref, lb_ref,
             l1n_ref, hout_ref, xlout_ref):
    agg = a0_ref[0] + a1_ref[0]
    t = _ssp(jnp.dot(agg, l2w_ref[...], preferred_element_type=jnp.float32)
             + l2b_ref[...])
    x2 = jnp.dot(t, lw_ref[...], preferred_element_type=jnp.float32) + lb_ref[...]
    hn = h_ref[...] + x2
    hout_ref[...] = hn
    xlout_ref[...] = jnp.dot(hn, l1n_ref[...], preferred_element_type=jnp.float32)


def _k4(aggp, h, l2w, l2b, lw, lb, l1n):
    return pl.pallas_call(
        _k4_body,
        grid=(NP_ // BN,),
        in_specs=[
            pl.BlockSpec((1, BN, H), lambda i: (0, i, 0)),
            pl.BlockSpec((1, BN, H), lambda i: (1, i, 0)),
            pl.BlockSpec((BN, H), lambda i: (i, 0)),
            pl.BlockSpec((H, H), lambda i: (0, 0)),
            pl.BlockSpec((1, H), lambda i: (0, 0)),
            pl.BlockSpec((H, H), lambda i: (0, 0)),
            pl.BlockSpec((1, H), lambda i: (0, 0)),
            pl.BlockSpec((H, H), lambda i: (0, 0)),
        ],
        out_specs=[
            pl.BlockSpec((BN, H), lambda i: (i, 0)),
            pl.BlockSpec((BN, H), lambda i: (i, 0)),
        ],
        out_shape=[
            jax.ShapeDtypeStruct((NP_, H), jnp.float32),
            jax.ShapeDtypeStruct((NP_, H), jnp.float32),
        ],
    )(aggp, aggp, h, l2w, l2b, lw, lb, l1n)


def _k5_body(b_ref, h_ref, pw_ref, pb_ref, out_ref, sums, counts):
    i = pl.program_id(0)

    @pl.when(i == 0)
    def _init():
        sums[...] = jnp.zeros_like(sums)
        counts[...] = jnp.zeros_like(counts)

    bb = b_ref[...]
    oh = (bb == lax.broadcasted_iota(jnp.int32, (1, NGRAPH), 1)).astype(jnp.float32)
    hb = h_ref[...]
    dn = (((0,), (0,)), ((), ()))
    sums[...] += lax.dot_general(oh, hb, dn, preferred_element_type=jnp.float32)
    counts[...] += lax.dot_general(oh, jnp.ones_like(hb), dn,
                                   preferred_element_type=jnp.float32)

    @pl.when(i == NP_ // BN - 1)
    def _fin():
        pooled = sums[...] / jnp.maximum(counts[...], 1.0)
        out_ref[...] = (jnp.dot(pooled, pw_ref[...],
                                preferred_element_type=jnp.float32) + pb_ref[...])


def _k5(batch_p, h, pool_W, pool_b):
    return pl.pallas_call(
        _k5_body,
        grid=(NP_ // BN,),
        in_specs=[
            pl.BlockSpec((BN, 1), lambda i: (i, 0)),
            pl.BlockSpec((BN, H), lambda i: (i, 0)),
            pl.BlockSpec((H, H), lambda i: (0, 0)),
            pl.BlockSpec((1, H), lambda i: (0, 0)),
        ],
        out_specs=pl.BlockSpec((NGRAPH, H), lambda i: (0, 0)),
        out_shape=jax.ShapeDtypeStruct((NGRAPH, H), jnp.float32),
        scratch_shapes=[
            pltpu.VMEM((NGRAPH, H), jnp.float32),
            pltpu.VMEM((NGRAPH, H), jnp.float32),
        ],
    )(batch_p, h, pool_W, pool_b)


def kernel(z, pos, batch, edge_index, emb, mlp_W1, mlp_b1, mlp_W2, mlp_b2,
           lin1_W, lin2_W, lin2_b, lin_W, lin_b, pool_W, pool_b):
    row = edge_index[0].astype(jnp.int32)
    col = edge_index[1].astype(jnp.int32)
    posf = pos.astype(jnp.float32)
    px = posf[:, 0] + 0.0
    py = posf[:, 1] + 0.0
    pz = posf[:, 2] + 0.0

    z_p = jnp.pad(z.astype(jnp.int32), (0, NP_ - N)).reshape(NP_, 1)
    batch_p = jnp.pad(batch.astype(jnp.int32), (0, NP_ - N),
                      constant_values=NGRAPH).reshape(NP_, 1)
    embp = jnp.pad(emb, ((0, VOCABP - emb.shape[0]), (0, 0)))
    w1p = jnp.pad(mlp_W1, ((0, 0), (0, NGP - NG), (0, 0)))
    zeros = jnp.zeros((NP_, H), jnp.float32)

    d2 = _k1(px, py, pz, row, col)
    d_arr, cut_arr = _k1b(d2.reshape(ER, H))
    dc = d_arr.reshape(E, 1)
    cutc = cut_arr.reshape(E, 1)

    h, xl = _k0(z_p, embp, lin1_W[0])
    for i in range(L):
        wf = _k2(dc, cutc, w1p[i], mlp_b1[i].reshape(1, H),
                 mlp_W2[i], mlp_b2[i].reshape(1, H))
        aggp = _k3(xl, wf, row, col, zeros)
        h, xl = _k4(aggp, h, lin2_W[i], lin2_b[i].reshape(1, H),
                    lin_W[i], lin_b[i].reshape(1, H), lin1_W[(i + 1) % L])

    return _k5(batch_p, h, pool_W, pool_b.reshape(1, H))


# speedup vs baseline: 5.2087x; 1.7917x over previous
import functools
import math

import jax
import jax.numpy as jnp
from jax import lax
from jax.experimental import pallas as pl
from jax.experimental.pallas import tpu as pltpu
from jax.experimental.pallas import tpu_sc as plsc

N = 10000
E = 320000
H = 128
NG = 50
NGP = 64
L = 6
CUTOFF = 10.0
NGRAPH = 16
VOCABP = 128
NP_ = 10240

NC = 2
NS = 16
NW = NC * NS
EW = E // NW
CH = 40
NCH = EW // CH
NPAIR = NCH // 2
RPT = NP_ // NS

_DELTA = CUTOFF / (NG - 1)
_COEFF = -0.5 / (_DELTA * _DELTA)
_LOG2 = math.log(2.0)


def _ssp(x):
    return jnp.maximum(x, 0.0) + jnp.log(1.0 + jnp.exp(-jnp.abs(x))) - _LOG2


def _k1_body(px_h, py_h, pz_h, row_h, col_h, d2_h,
             px_v, py_v, pz_v, row_v, col_v, d2_v):
    c = lax.axis_index("c")
    s = lax.axis_index("s")
    wid = s * NC + c
    base = wid * EW
    pltpu.sync_copy(px_h, px_v)
    pltpu.sync_copy(py_h, py_v)
    pltpu.sync_copy(pz_h, pz_v)
    pltpu.sync_copy(row_h.at[pl.ds(base, EW)], row_v)
    pltpu.sync_copy(col_h.at[pl.ds(base, EW)], col_v)

    def body(i, carry):
        sl = pl.ds(i * 16, 16)
        r = row_v[sl]
        cc = col_v[sl]
        dx = plsc.load_gather(px_v, [r]) - plsc.load_gather(px_v, [cc])
        dy = plsc.load_gather(py_v, [r]) - plsc.load_gather(py_v, [cc])
        dz = plsc.load_gather(pz_v, [r]) - plsc.load_gather(pz_v, [cc])
        d2_v[sl] = dx * dx + dy * dy + dz * dz
        return carry

    lax.fori_loop(0, EW // 16, body, 0)
    pltpu.sync_copy(d2_v, d2_h.at[pl.ds(base, EW)])


_k1 = functools.partial(
    pl.kernel,
    mesh=plsc.VectorSubcoreMesh(core_axis_name="c", subcore_axis_name="s"),
    compiler_params=pltpu.CompilerParams(needs_layout_passes=False),
    out_type=jax.ShapeDtypeStruct((E,), jnp.float32),
    scratch_types=[
        pltpu.VMEM((N,), jnp.float32),
        pltpu.VMEM((N,), jnp.float32),
        pltpu.VMEM((N,), jnp.float32),
        pltpu.VMEM((EW,), jnp.int32),
        pltpu.VMEM((EW,), jnp.int32),
        pltpu.VMEM((EW,), jnp.float32),
    ],
)(_k1_body)


def _k3_body(xl_h, wf_h, row_h, col_h, zeros_h, out_h,
             agg_sp, row1d, cb0, cb1, r0, w0, r1, w1,
             sg0, sw0, sg1, sw1, sc0, sc1):
    c = lax.axis_index("c")
    s = lax.axis_index("s")
    wid = s * NC + c
    base = wid * EW
    pltpu.sync_copy(zeros_h.at[pl.ds(s * RPT, RPT)],
                    agg_sp.at[pl.ds(s * RPT, RPT)])
    pltpu.sync_copy(row_h.at[pl.ds(base, EW)], row1d)
    plsc.subcore_barrier()

    def issue(ci, rv, wv, cb, sg, sw, sc):
        pltpu.async_copy(xl_h.at[row1d.at[pl.ds(ci * CH, CH)]], rv, sg)
        pltpu.async_copy(wf_h.at[pl.ds(base + ci * CH, CH)], wv, sw)
        pltpu.async_copy(col_h.at[pl.ds(base + ci * CH, CH)], cb, sc)

    def wait_in(rv, wv, cb, sg, sw, sc):
        pltpu.make_async_copy(xl_h.at[pl.ds(0, CH)], rv, sg).wait()
        pltpu.make_async_copy(wf_h.at[pl.ds(0, CH)], wv, sw).wait()
        pltpu.make_async_copy(col_h.at[pl.ds(0, CH)], cb, sc).wait()

    def mul_scatter(rv, wv, cb):
        def mrow(r, carry):
            for j in range(8):
                sl = pl.ds(j * 16, 16)
                rv[r, sl] = rv[r, sl] * wv[r, sl]
            return carry

        lax.fori_loop(0, CH, mrow, 0)
        pltpu.sync_copy(rv, agg_sp.at[cb], add=True)

    issue(0, r0, w0, cb0, sg0, sw0, sc0)

    def pair(k, carry):
        c0 = 2 * k
        issue(c0 + 1, r1, w1, cb1, sg1, sw1, sc1)
        wait_in(r0, w0, cb0, sg0, sw0, sc0)
        mul_scatter(r0, w0, cb0)

        @pl.when(k + 1 < NPAIR)
        def _next():
            issue(c0 + 2, r0, w0, cb0, sg0, sw0, sc0)

        wait_in(r1, w1, cb1, sg1, sw1, sc1)
        mul_scatter(r1, w1, cb1)
        return carry

    lax.fori_loop(0, NPAIR, pair, 0)

    plsc.subcore_barrier()
    pltpu.sync_copy(agg_sp.at[pl.ds(s * RPT, RPT)],
                    out_h.at[c, pl.ds(s * RPT, RPT)])


_k3 = functools.partial(
    pl.kernel,
    mesh=plsc.VectorSubcoreMesh(core_axis_name="c", subcore_axis_name="s"),
    out_type=jax.ShapeDtypeStruct((NC, NP_, H), jnp.float32),
    scratch_types=[
        pltpu.VMEM_SHARED((NP_, H), jnp.float32),
        pltpu.VMEM((EW,), jnp.int32),
        pltpu.VMEM((CH,), jnp.int32),
        pltpu.VMEM((CH,), jnp.int32),
        pltpu.VMEM((CH, H), jnp.float32),
        pltpu.VMEM((CH, H), jnp.float32),
        pltpu.VMEM((CH, H), jnp.float32),
        pltpu.VMEM((CH, H), jnp.float32),
        pltpu.SemaphoreType.DMA,
        pltpu.SemaphoreType.DMA,
        pltpu.SemaphoreType.DMA,
        pltpu.SemaphoreType.DMA,
        pltpu.SemaphoreType.DMA,
        pltpu.SemaphoreType.DMA,
    ],
)(_k3_body)


ER = E // H
BR = 2500


def _k1b_body(d2_ref, d_ref, cut_ref):
    d2 = d2_ref[...]
    d = jnp.sqrt(d2 + 1e-12)
    d_ref[...] = d
    cut_ref[...] = 0.5 * (jnp.cos(d * (math.pi / CUTOFF)) + 1.0)


def _k1b(d2m):
    return pl.pallas_call(
        _k1b_body,
        grid=(ER // BR,),
        in_specs=[pl.BlockSpec((BR, H), lambda i: (i, 0))],
        out_specs=[
            pl.BlockSpec((BR, H), lambda i: (i, 0)),
            pl.BlockSpec((BR, H), lambda i: (i, 0)),
        ],
        out_shape=[
            jax.ShapeDtypeStruct((ER, H), jnp.float32),
            jax.ShapeDtypeStruct((ER, H), jnp.float32),
        ],
    )(d2m)


BE = 2560


def _k2_body(d_ref, cut_ref, w1_ref, b1_ref, w2_ref, b2_ref, wf_ref):
    d = d_ref[...]
    offs = _DELTA * lax.broadcasted_iota(jnp.int32, (1, NGP), 1).astype(jnp.float32)
    g = jnp.exp(_COEFF * (d - offs) ** 2)
    t = jnp.dot(g, w1_ref[...], preferred_element_type=jnp.float32) + b1_ref[...]
    t = _ssp(t)
    t = jnp.dot(t, w2_ref[...], preferred_element_type=jnp.float32) + b2_ref[...]
    wf_ref[...] = t * cut_ref[...]


def _k2(dc, cutc, w1p, b1, w2, b2):
    return pl.pallas_call(
        _k2_body,
        grid=(E // BE,),
        in_specs=[
            pl.BlockSpec((BE, 1), lambda i: (i, 0)),
            pl.BlockSpec((BE, 1), lambda i: (i, 0)),
            pl.BlockSpec((NGP, H), lambda i: (0, 0)),
            pl.BlockSpec((1, H), lambda i: (0, 0)),
            pl.BlockSpec((H, H), lambda i: (0, 0)),
            pl.BlockSpec((1, H), lambda i: (0, 0)),
        ],
        out_specs=pl.BlockSpec((BE, H), lambda i: (i, 0)),
        out_shape=jax.ShapeDtypeStruct((E, H), jnp.float32),
    )(dc, cutc, w1p, b1, w2, b2)


BN = 2560


def _k0_body(z_ref, emb_ref, lin1_ref, h_ref, xl_ref):
    zb = z_ref[...]
    oh = (zb == lax.broadcasted_iota(jnp.int32, (1, VOCABP), 1)).astype(jnp.float32)
    h = jnp.dot(oh, emb_ref[...], preferred_element_type=jnp.float32)
    h_ref[...] = h
    xl_ref[...] = jnp.dot(h, lin1_ref[...], preferred_element_type=jnp.float32)


def _k0(z_p, embp, lin1_0):
    return pl.pallas_call(
        _k0_body,
        grid=(NP_ // BN,),
        in_specs=[
            pl.BlockSpec((BN, 1), lambda i: (i, 0)),
            pl.BlockSpec((VOCABP, H), lambda i: (0, 0)),
            pl.BlockSpec((H, H), lambda i: (0, 0)),
        ],
        out_specs=[
            pl.BlockSpec((BN, H), lambda i: (i, 0)),
            pl.BlockSpec((BN, H), lambda i: (i, 0)),
        ],
        out_shape=[
            jax.ShapeDtypeStruct((NP_, H), jnp.float32),
            jax.ShapeDtypeStruct((NP_, H), jnp.float32),
        ],
    )(z_p, embp, lin1_0)


def _k4_body(a0_ref, a1_ref, h_ref, l2w_ref, l2b_ref, lw_ref, lb_ref,
             l1n_ref, hout_ref, xlout_ref):
    agg = a0_ref[0] + a1_ref[0]
    t = _ssp(jnp.dot(agg, l2w_ref[...], preferred_element_type=jnp.float32)
             + l2b_ref[...])
    x2 = jnp.dot(t, lw_ref[...], preferred_element_type=jnp.float32) + lb_ref[...]
    hn = h_ref[...] + x2
    hout_ref[...] = hn
    xlout_ref[...] = jnp.dot(hn, l1n_ref[...], preferred_element_type=jnp.float32)


def _k4(aggp, h, l2w, l2b, lw, lb, l1n):
    return pl.pallas_call(
        _k4_body,
        grid=(NP_ // BN,),
        in_specs=[
            pl.BlockSpec((1, BN, H), lambda i: (0, i, 0)),
            pl.BlockSpec((1, BN, H), lambda i: (1, i, 0)),
            pl.BlockSpec((BN, H), lambda i: (i, 0)),
            pl.BlockSpec((H, H), lambda i: (0, 0)),
            pl.BlockSpec((1, H), lambda i: (0, 0)),
            pl.BlockSpec((H, H), lambda i: (0, 0)),
            pl.BlockSpec((1, H), lambda i: (0, 0)),
            pl.BlockSpec((H, H), lambda i: (0, 0)),
        ],
        out_specs=[
            pl.BlockSpec((BN, H), lambda i: (i, 0)),
            pl.BlockSpec((BN, H), lambda i: (i, 0)),
        ],
        out_shape=[
            jax.ShapeDtypeStruct((NP_, H), jnp.float32),
            jax.ShapeDtypeStruct((NP_, H), jnp.float32),
        ],
    )(aggp, aggp, h, l2w, l2b, lw, lb, l1n)


def _k5_body(b_ref, h_ref, pw_ref, pb_ref, out_ref, sums, counts):
    i = pl.program_id(0)

    @pl.when(i == 0)
    def _init():
        sums[...] = jnp.zeros_like(sums)
        counts[...] = jnp.zeros_like(counts)

    bb = b_ref[...]
    oh = (bb == lax.broadcasted_iota(jnp.int32, (1, NGRAPH), 1)).astype(jnp.float32)
    hb = h_ref[...]
    dn = (((0,), (0,)), ((), ()))
    sums[...] += lax.dot_general(oh, hb, dn, preferred_element_type=jnp.float32)
    counts[...] += lax.dot_general(oh, jnp.ones_like(hb), dn,
                                   preferred_element_type=jnp.float32)

    @pl.when(i == NP_ // BN - 1)
    def _fin():
        pooled = sums[...] / jnp.maximum(counts[...], 1.0)
        out_ref[...] = (jnp.dot(pooled, pw_ref[...],
                                preferred_element_type=jnp.float32) + pb_ref[...])


def _k5(batch_p, h, pool_W, pool_b):
    return pl.pallas_call(
        _k5_body,
        grid=(NP_ // BN,),
        in_specs=[
            pl.BlockSpec((BN, 1), lambda i: (i, 0)),
            pl.BlockSpec((BN, H), lambda i: (i, 0)),
            pl.BlockSpec((H, H), lambda i: (0, 0)),
            pl.BlockSpec((1, H), lambda i: (0, 0)),
        ],
        out_specs=pl.BlockSpec((NGRAPH, H), lambda i: (0, 0)),
        out_shape=jax.ShapeDtypeStruct((NGRAPH, H), jnp.float32),
        scratch_shapes=[
            pltpu.VMEM((NGRAPH, H), jnp.float32),
            pltpu.VMEM((NGRAPH, H), jnp.float32),
        ],
    )(batch_p, h, pool_W, pool_b)


def kernel(z, pos, batch, edge_index, emb, mlp_W1, mlp_b1, mlp_W2, mlp_b2,
           lin1_W, lin2_W, lin2_b, lin_W, lin_b, pool_W, pool_b):
    row = edge_index[0].astype(jnp.int32)
    col = edge_index[1].astype(jnp.int32)
    posf = pos.astype(jnp.float32)
    px = posf[:, 0] + 0.0
    py = posf[:, 1] + 0.0
    pz = posf[:, 2] + 0.0

    z_p = jnp.pad(z.astype(jnp.int32), (0, NP_ - N)).reshape(NP_, 1)
    batch_p = jnp.pad(batch.astype(jnp.int32), (0, NP_ - N),
                      constant_values=NGRAPH).reshape(NP_, 1)
    embp = jnp.pad(emb, ((0, VOCABP - emb.shape[0]), (0, 0)))
    w1p = jnp.pad(mlp_W1, ((0, 0), (0, NGP - NG), (0, 0)))
    zeros = jnp.zeros((NP_, H), jnp.float32)

    d2 = _k1(px, py, pz, row, col)
    d_arr, cut_arr = _k1b(d2.reshape(ER, H))
    dc = d_arr.reshape(E, 1)
    cutc = cut_arr.reshape(E, 1)

    h, xl = _k0(z_p, embp, lin1_W[0])
    for i in range(L):
        wf = _k2(dc, cutc, w1p[i], mlp_b1[i].reshape(1, H),
                 mlp_W2[i], mlp_b2[i].reshape(1, H))
        aggp = _k3(xl, wf, row, col, zeros)
        h, xl = _k4(aggp, h, lin2_W[i], lin2_b[i].reshape(1, H),
                    lin_W[i], lin_b[i].reshape(1, H), lin1_W[(i + 1) % L])

    return _k5(batch_p, h, pool_W, pool_b.reshape(1, H))


# speedup vs baseline: 5.4143x; 1.0395x over previous
import functools
import math

import jax
import jax.numpy as jnp
from jax import lax
from jax.experimental import pallas as pl
from jax.experimental.pallas import tpu as pltpu
from jax.experimental.pallas import tpu_sc as plsc

N = 10000
E = 320000
H = 128
NG = 50
NGP = 64
L = 6
CUTOFF = 10.0
NGRAPH = 16
VOCABP = 128
NP_ = 10240

NC = 2
NS = 16
NW = NC * NS
EW = E // NW
CH = 40
NCH = EW // CH
NPAIR = NCH // 2
RPT = NP_ // NS

_DELTA = CUTOFF / (NG - 1)
_COEFF = -0.5 / (_DELTA * _DELTA)
_LOG2 = math.log(2.0)


def _ssp(x):
    return jnp.maximum(x, 0.0) + jnp.log(1.0 + jnp.exp(-jnp.abs(x))) - _LOG2


def _k1_body(px_h, py_h, pz_h, row_h, col_h, d2_h,
             px_v, py_v, pz_v, row_v, col_v, d2_v):
    c = lax.axis_index("c")
    s = lax.axis_index("s")
    wid = s * NC + c
    base = wid * EW
    pltpu.sync_copy(px_h, px_v)
    pltpu.sync_copy(py_h, py_v)
    pltpu.sync_copy(pz_h, pz_v)
    pltpu.sync_copy(row_h.at[pl.ds(base, EW)], row_v)
    pltpu.sync_copy(col_h.at[pl.ds(base, EW)], col_v)

    def body(i, carry):
        sl = pl.ds(i * 16, 16)
        r = row_v[sl]
        cc = col_v[sl]
        dx = plsc.load_gather(px_v, [r]) - plsc.load_gather(px_v, [cc])
        dy = plsc.load_gather(py_v, [r]) - plsc.load_gather(py_v, [cc])
        dz = plsc.load_gather(pz_v, [r]) - plsc.load_gather(pz_v, [cc])
        d2_v[sl] = dx * dx + dy * dy + dz * dz
        return carry

    lax.fori_loop(0, EW // 16, body, 0)
    pltpu.sync_copy(d2_v, d2_h.at[pl.ds(base, EW)])


_k1 = functools.partial(
    pl.kernel,
    mesh=plsc.VectorSubcoreMesh(core_axis_name="c", subcore_axis_name="s"),
    compiler_params=pltpu.CompilerParams(needs_layout_passes=False),
    out_type=jax.ShapeDtypeStruct((E,), jnp.float32),
    scratch_types=[
        pltpu.VMEM((N,), jnp.float32),
        pltpu.VMEM((N,), jnp.float32),
        pltpu.VMEM((N,), jnp.float32),
        pltpu.VMEM((EW,), jnp.int32),
        pltpu.VMEM((EW,), jnp.int32),
        pltpu.VMEM((EW,), jnp.float32),
    ],
)(_k1_body)


def _k3_body(xl_h, wf_h, row_h, col_h, zeros_h, out_h, agg_sp,
             ir0, ir1, ir2, cb0, cb1, cb2, r0, r1, r2, w0, w1, w2,
             si0, si1, si2, sc0, sc1, sc2, sg0, sg1, sg2,
             sw0, sw1, sw2, ss0, ss1, ss2):
    c = lax.axis_index("c")
    s = lax.axis_index("s")
    wid = s * NC + c
    base = wid * EW
    irs = (ir0, ir1, ir2)
    cbs = (cb0, cb1, cb2)
    rs = (r0, r1, r2)
    ws = (w0, w1, w2)
    sis = (si0, si1, si2)
    scs = (sc0, sc1, sc2)
    sgs = (sg0, sg1, sg2)
    sws = (sw0, sw1, sw2)
    sss = (ss0, ss1, ss2)

    pltpu.sync_copy(zeros_h.at[pl.ds(s * RPT, RPT)],
                    agg_sp.at[pl.ds(s * RPT, RPT)])
    plsc.subcore_barrier()

    def issue_row_idx(ci, b):
        pltpu.async_copy(row_h.at[pl.ds(base + ci * CH, CH)], irs[b], sis[b])

    def issue_gather(ci, b):
        pltpu.make_async_copy(row_h.at[pl.ds(0, CH)], irs[b], sis[b]).wait()
        pltpu.async_copy(xl_h.at[irs[b]], rs[b], sgs[b])
        pltpu.async_copy(wf_h.at[pl.ds(base + ci * CH, CH)], ws[b], sws[b])

    def step(ci, b):
        bp1 = (b + 1) % 3
        bp2 = (b + 2) % 3

        @pl.when(ci + 2 < NCH)
        def _pf_idx():
            issue_row_idx(ci + 2, bp2)

        @pl.when(ci + 1 < NCH)
        def _pf_gather():
            @pl.when(ci + 1 >= 3)
            def _drain():
                pltpu.make_async_copy(zeros_h.at[pl.ds(0, CH)],
                                      rs[bp1], sss[bp1]).wait()

            pltpu.async_copy(col_h.at[pl.ds(base + (ci + 1) * CH, CH)],
                             cbs[bp1], scs[bp1])
            issue_gather(ci + 1, bp1)

        pltpu.make_async_copy(zeros_h.at[pl.ds(0, CH)], rs[b], sgs[b]).wait()
        pltpu.make_async_copy(zeros_h.at[pl.ds(0, CH)], ws[b], sws[b]).wait()

        def mrow(r, carry):
            for j in range(8):
                sl = pl.ds(j * 16, 16)
                rs[b][r, sl] = rs[b][r, sl] * ws[b][r, sl]
            return carry

        lax.fori_loop(0, CH, mrow, 0)
        pltpu.make_async_copy(col_h.at[pl.ds(0, CH)], cbs[b], scs[b]).wait()
        pltpu.async_copy(rs[b], agg_sp.at[cbs[b]], sss[b], add=True)

    issue_row_idx(0, 0)
    issue_row_idx(1, 1)
    pltpu.async_copy(col_h.at[pl.ds(base, CH)], cbs[0], scs[0])
    issue_gather(0, 0)

    def triple(t, carry):
        c0 = 3 * t
        step(c0, 0)
        step(c0 + 1, 1)
        step(c0 + 2, 2)
        return carry

    lax.fori_loop(0, NCH // 3, triple, 0)
    step(NCH - 1, 0)

    pltpu.make_async_copy(zeros_h.at[pl.ds(0, CH)], rs[1], sss[1]).wait()
    pltpu.make_async_copy(zeros_h.at[pl.ds(0, CH)], rs[2], sss[2]).wait()
    pltpu.make_async_copy(zeros_h.at[pl.ds(0, CH)], rs[0], sss[0]).wait()

    plsc.subcore_barrier()
    pltpu.sync_copy(agg_sp.at[pl.ds(s * RPT, RPT)],
                    out_h.at[c, pl.ds(s * RPT, RPT)])


_k3 = functools.partial(
    pl.kernel,
    mesh=plsc.VectorSubcoreMesh(core_axis_name="c", subcore_axis_name="s"),
    out_type=jax.ShapeDtypeStruct((NC, NP_, H), jnp.float32),
    scratch_types=[
        pltpu.VMEM_SHARED((NP_, H), jnp.float32),
        pltpu.VMEM((CH,), jnp.int32),
        pltpu.VMEM((CH,), jnp.int32),
        pltpu.VMEM((CH,), jnp.int32),
        pltpu.VMEM((CH,), jnp.int32),
        pltpu.VMEM((CH,), jnp.int32),
        pltpu.VMEM((CH,), jnp.int32),
        pltpu.VMEM((CH, H), jnp.float32),
        pltpu.VMEM((CH, H), jnp.float32),
        pltpu.VMEM((CH, H), jnp.float32),
        pltpu.VMEM((CH, H), jnp.float32),
        pltpu.VMEM((CH, H), jnp.float32),
        pltpu.VMEM((CH, H), jnp.float32),
    ] + [pltpu.SemaphoreType.DMA] * 15,
)(_k3_body)


ER = E // H
BR = 2500


def _k1b_body(d2_ref, d_ref, cut_ref):
    d2 = d2_ref[...]
    d = jnp.sqrt(d2 + 1e-12)
    d_ref[...] = d
    cut_ref[...] = 0.5 * (jnp.cos(d * (math.pi / CUTOFF)) + 1.0)


def _k1b(d2m):
    return pl.pallas_call(
        _k1b_body,
        grid=(ER // BR,),
        in_specs=[pl.BlockSpec((BR, H), lambda i: (i, 0))],
        out_specs=[
            pl.BlockSpec((BR, H), lambda i: (i, 0)),
            pl.BlockSpec((BR, H), lambda i: (i, 0)),
        ],
        out_shape=[
            jax.ShapeDtypeStruct((ER, H), jnp.float32),
            jax.ShapeDtypeStruct((ER, H), jnp.float32),
        ],
    )(d2m)


BE = 2560


def _k2_body(d_ref, cut_ref, w1_ref, b1_ref, w2_ref, b2_ref, wf_ref):
    d = d_ref[...]
    offs = _DELTA * lax.broadcasted_iota(jnp.int32, (1, NGP), 1).astype(jnp.float32)
    g = jnp.exp(_COEFF * (d - offs) ** 2)
    t = jnp.dot(g, w1_ref[...], preferred_element_type=jnp.float32) + b1_ref[...]
    t = _ssp(t)
    t = jnp.dot(t, w2_ref[...], preferred_element_type=jnp.float32) + b2_ref[...]
    wf_ref[...] = t * cut_ref[...]


def _k2(dc, cutc, w1p, b1, w2, b2):
    return pl.pallas_call(
        _k2_body,
        grid=(E // BE,),
        in_specs=[
            pl.BlockSpec((BE, 1), lambda i: (i, 0)),
            pl.BlockSpec((BE, 1), lambda i: (i, 0)),
            pl.BlockSpec((NGP, H), lambda i: (0, 0)),
            pl.BlockSpec((1, H), lambda i: (0, 0)),
            pl.BlockSpec((H, H), lambda i: (0, 0)),
            pl.BlockSpec((1, H), lambda i: (0, 0)),
        ],
        out_specs=pl.BlockSpec((BE, H), lambda i: (i, 0)),
        out_shape=jax.ShapeDtypeStruct((E, H), jnp.float32),
    )(dc, cutc, w1p, b1, w2, b2)


BN = 2560


def _k0_body(z_ref, emb_ref, lin1_ref, h_ref, xl_ref):
    zb = z_ref[...]
    oh = (zb == lax.broadcasted_iota(jnp.int32, (1, VOCABP), 1)).astype(jnp.float32)
    h = jnp.dot(oh, emb_ref[...], preferred_element_type=jnp.float32)
    h_ref[...] = h
    xl_ref[...] = jnp.dot(h, lin1_ref[...], preferred_element_type=jnp.float32)


def _k0(z_p, embp, lin1_0):
    return pl.pallas_call(
        _k0_body,
        grid=(NP_ // BN,),
        in_specs=[
            pl.BlockSpec((BN, 1), lambda i: (i, 0)),
            pl.BlockSpec((VOCABP, H), lambda i: (0, 0)),
            pl.BlockSpec((H, H), lambda i: (0, 0)),
        ],
        out_specs=[
            pl.BlockSpec((BN, H), lambda i: (i, 0)),
            pl.BlockSpec((BN, H), lambda i: (i, 0)),
        ],
        out_shape=[
            jax.ShapeDtypeStruct((NP_, H), jnp.float32),
            jax.ShapeDtypeStruct((NP_, H), jnp.float32),
        ],
    )(z_p, embp, lin1_0)


def _k4_body(a0_ref, a1_ref, h_ref, l2w_ref, l2b_ref, lw_ref, lb_ref,
             l1n_ref, hout_ref, xlout_ref):
    agg = a0_ref[0] + a1_ref[0]
    t = _ssp(jnp.dot(agg, l2w_ref[...], preferred_element_type=jnp.float32)
             + l2b_ref[...])
    x2 = jnp.dot(t, lw_ref[...], preferred_element_type=jnp.float32) + lb_ref[...]
    hn = h_ref[...] + x2
    hout_ref[...] = hn
    xlout_ref[...] = jnp.dot(hn, l1n_ref[...], preferred_element_type=jnp.float32)


def _k4(aggp, h, l2w, l2b, lw, lb, l1n):
    return pl.pallas_call(
        _k4_body,
        grid=(NP_ // BN,),
        in_specs=[
            pl.BlockSpec((1, BN, H), lambda i: (0, i, 0)),
            pl.BlockSpec((1, BN, H), lambda i: (1, i, 0)),
            pl.BlockSpec((BN, H), lambda i: (i, 0)),
            pl.BlockSpec((H, H), lambda i: (0, 0)),
            pl.BlockSpec((1, H), lambda i: (0, 0)),
            pl.BlockSpec((H, H), lambda i: (0, 0)),
            pl.BlockSpec((1, H), lambda i: (0, 0)),
            pl.BlockSpec((H, H), lambda i: (0, 0)),
        ],
        out_specs=[
            pl.BlockSpec((BN, H), lambda i: (i, 0)),
            pl.BlockSpec((BN, H), lambda i: (i, 0)),
        ],
        out_shape=[
            jax.ShapeDtypeStruct((NP_, H), jnp.float32),
            jax.ShapeDtypeStruct((NP_, H), jnp.float32),
        ],
    )(aggp, aggp, h, l2w, l2b, lw, lb, l1n)


def _k5_body(b_ref, h_ref, pw_ref, pb_ref, out_ref, sums, counts):
    i = pl.program_id(0)

    @pl.when(i == 0)
    def _init():
        sums[...] = jnp.zeros_like(sums)
        counts[...] = jnp.zeros_like(counts)

    bb = b_ref[...]
    oh = (bb == lax.broadcasted_iota(jnp.int32, (1, NGRAPH), 1)).astype(jnp.float32)
    hb = h_ref[...]
    dn = (((0,), (0,)), ((), ()))
    sums[...] += lax.dot_general(oh, hb, dn, preferred_element_type=jnp.float32)
    counts[...] += lax.dot_general(oh, jnp.ones_like(hb), dn,
                                   preferred_element_type=jnp.float32)

    @pl.when(i == NP_ // BN - 1)
    def _fin():
        pooled = sums[...] / jnp.maximum(counts[...], 1.0)
        out_ref[...] = (jnp.dot(pooled, pw_ref[...],
                                preferred_element_type=jnp.float32) + pb_ref[...])


def _k5(batch_p, h, pool_W, pool_b):
    return pl.pallas_call(
        _k5_body,
        grid=(NP_ // BN,),
        in_specs=[
            pl.BlockSpec((BN, 1), lambda i: (i, 0)),
            pl.BlockSpec((BN, H), lambda i: (i, 0)),
            pl.BlockSpec((H, H), lambda i: (0, 0)),
            pl.BlockSpec((1, H), lambda i: (0, 0)),
        ],
        out_specs=pl.BlockSpec((NGRAPH, H), lambda i: (0, 0)),
        out_shape=jax.ShapeDtypeStruct((NGRAPH, H), jnp.float32),
        scratch_shapes=[
            pltpu.VMEM((NGRAPH, H), jnp.float32),
            pltpu.VMEM((NGRAPH, H), jnp.float32),
        ],
    )(batch_p, h, pool_W, pool_b)


def kernel(z, pos, batch, edge_index, emb, mlp_W1, mlp_b1, mlp_W2, mlp_b2,
           lin1_W, lin2_W, lin2_b, lin_W, lin_b, pool_W, pool_b):
    row = edge_index[0].astype(jnp.int32)
    col = edge_index[1].astype(jnp.int32)
    posf = pos.astype(jnp.float32)
    px = posf[:, 0] + 0.0
    py = posf[:, 1] + 0.0
    pz = posf[:, 2] + 0.0

    z_p = jnp.pad(z.astype(jnp.int32), (0, NP_ - N)).reshape(NP_, 1)
    batch_p = jnp.pad(batch.astype(jnp.int32), (0, NP_ - N),
                      constant_values=NGRAPH).reshape(NP_, 1)
    embp = jnp.pad(emb, ((0, VOCABP - emb.shape[0]), (0, 0)))
    w1p = jnp.pad(mlp_W1, ((0, 0), (0, NGP - NG), (0, 0)))
    zeros = jnp.zeros((NP_, H), jnp.float32)

    d2 = _k1(px, py, pz, row, col)
    d_arr, cut_arr = _k1b(d2.reshape(ER, H))
    dc = d_arr.reshape(E, 1)
    cutc = cut_arr.reshape(E, 1)

    h, xl = _k0(z_p, embp, lin1_W[0])
    for i in range(L):
        wf = _k2(dc, cutc, w1p[i], mlp_b1[i].reshape(1, H),
                 mlp_W2[i], mlp_b2[i].reshape(1, H))
        aggp = _k3(xl, wf, row, col, zeros)
        h, xl = _k4(aggp, h, lin2_W[i], lin2_b[i].reshape(1, H),
                    lin_W[i], lin_b[i].reshape(1, H), lin1_W[(i + 1) % L])

    return _k5(batch_p, h, pool_W, pool_b.reshape(1, H))
